# baseline (speedup 1.0000x reference)
import jax
import jax.numpy as jnp
from jax.experimental import pallas as pl
from jax.experimental.pallas import tpu as pltpu

B = 128
V = 196
D = 768
TV = 16
NSTEPS = 13
VP = NSTEPS * TV

NEG_INF = float("-inf")


def _sim_kernel(l_ref, vis_ref, loss_ref, idx_ref, m0_ref, m1_ref, vsum_ref):
    i = pl.program_id(0)
    lmat = l_ref[...]

    @pl.when(i == 0)
    def _():
        m0_ref[...] = jnp.full((B, B), NEG_INF, jnp.float32)
        m1_ref[...] = jnp.full((B, B), NEG_INF, jnp.float32)

    vsum_ref[pl.ds(i * TV, TV), :] = jnp.sum(vis_ref[...], axis=1)

    m0 = m0_ref[...]
    m1 = m1_ref[...]
    for j in range(TV):
        s = jax.lax.dot_general(
            vis_ref[j], lmat, (((1,), (1,)), ((), ())),
            preferred_element_type=jnp.float32)
        s = jnp.where(i * TV + j < V, s, NEG_INF)
        hi = jnp.maximum(m0, s)
        lo = jnp.minimum(m0, s)
        m0 = hi
        m1 = jnp.maximum(m1, lo)
    m0_ref[...] = m0
    m1_ref[...] = m1

    @pl.when(i == NSTEPS - 1)
    def _():
        anchor = jax.lax.dot_general(
            vsum_ref[...], lmat, (((1,), (1,)), ((), ())),
            preferred_element_type=jnp.float32)
        iota_v = jax.lax.broadcasted_iota(jnp.int32, (VP, B), 0)
        anchor = jnp.where(iota_v < V, anchor, NEG_INF)
        colmax = jnp.max(anchor, axis=0, keepdims=True)
        idx_ref[...] = jnp.min(
            jnp.where(anchor == colmax, iota_v, V), axis=0, keepdims=True)

        m0m = m0_ref[...]
        m1m = m1_ref[...]
        mcol = jnp.maximum(jnp.max(m0m, axis=0, keepdims=True),
                           jnp.max(m1m, axis=0, keepdims=True))
        e0 = jnp.exp(m0m - mcol)
        e1 = jnp.exp(m1m - mcol)
        lanes = jax.lax.broadcasted_iota(jnp.int32, (B, B), 1)
        rows = jax.lax.broadcasted_iota(jnp.int32, (B, B), 0)
        diag = lanes == rows
        z = (jnp.sum(e0, axis=0) + jnp.sum(e1, axis=0)
             - jnp.sum(jnp.where(diag, e1, 0.0), axis=0))
        logz = jnp.log(z) + mcol[0]
        diag0 = jnp.sum(jnp.where(diag, m0m, 0.0), axis=0)
        loss_ref[...] = (jnp.sum(logz - diag0) * (1.0 / B)).reshape(1, 1)


def _gather_kernel(idx_ref, vis_ref, emb_ref, sem):
    copies = []
    for b in range(B):
        c = pltpu.make_async_copy(
            vis_ref.at[pl.ds(idx_ref[b], 1), b, :], emb_ref.at[b], sem)
        c.start()
        copies.append(c)
    for c in copies:
        c.wait()


@jax.jit
def kernel(fusion_fs, lan_fs):
    vis_t = jnp.transpose(fusion_fs, (1, 0, 2))
    lmat = lan_fs.reshape(B, D)

    loss2d, idx2d = pl.pallas_call(
        _sim_kernel,
        grid=(NSTEPS,),
        in_specs=[
            pl.BlockSpec((B, D), lambda i: (0, 0)),
            pl.BlockSpec((TV, B, D), lambda i: (i, 0, 0)),
        ],
        out_specs=[
            pl.BlockSpec((1, 1), lambda i: (0, 0)),
            pl.BlockSpec((1, B), lambda i: (0, 0)),
        ],
        out_shape=[
            jax.ShapeDtypeStruct((1, 1), jnp.float32),
            jax.ShapeDtypeStruct((1, B), jnp.int32),
        ],
        scratch_shapes=[
            pltpu.VMEM((B, B), jnp.float32),
            pltpu.VMEM((B, B), jnp.float32),
            pltpu.VMEM((VP, D), jnp.float32),
        ],
    )(lmat, vis_t)

    emb = pl.pallas_call(
        _gather_kernel,
        grid_spec=pltpu.PrefetchScalarGridSpec(
            num_scalar_prefetch=1,
            grid=(1,),
            in_specs=[pl.BlockSpec(memory_space=pl.ANY)],
            out_specs=pl.BlockSpec((B, 1, D), lambda i, idx: (0, 0, 0)),
            scratch_shapes=[pltpu.SemaphoreType.DMA],
        ),
        out_shape=jax.ShapeDtypeStruct((B, 1, D), jnp.float32),
    )(idx2d[0], vis_t)

    return loss2d[0, 0], emb


# speedup vs baseline: 1.0291x; 1.0291x over previous
import jax
import jax.numpy as jnp
from jax.experimental import pallas as pl
from jax.experimental.pallas import tpu as pltpu

B = 128
V = 196
D = 768
TV = 24
NSTEPS = 9
VP = NSTEPS * TV

NEG_INF = float("-inf")


def _sim_kernel(l_ref, vis_ref, loss_ref, idx_ref, m0_ref, m1_ref, vsum_ref):
    i = pl.program_id(0)
    lmat = l_ref[...]

    @pl.when(i == 0)
    def _():
        m0_ref[...] = jnp.full((B, B), NEG_INF, jnp.float32)
        m1_ref[...] = jnp.full((B, B), NEG_INF, jnp.float32)

    vsum_ref[pl.ds(i * TV, TV), :] = jnp.sum(vis_ref[...], axis=1)

    m0 = m0_ref[...]
    m1 = m1_ref[...]
    for j in range(TV):
        s = jax.lax.dot_general(
            vis_ref[j], lmat, (((1,), (1,)), ((), ())),
            preferred_element_type=jnp.float32)
        s = jnp.where(i * TV + j < V, s, NEG_INF)
        hi = jnp.maximum(m0, s)
        lo = jnp.minimum(m0, s)
        m0 = hi
        m1 = jnp.maximum(m1, lo)
    m0_ref[...] = m0
    m1_ref[...] = m1

    @pl.when(i == NSTEPS - 1)
    def _():
        anchor = jax.lax.dot_general(
            vsum_ref[...], lmat, (((1,), (1,)), ((), ())),
            preferred_element_type=jnp.float32)
        iota_v = jax.lax.broadcasted_iota(jnp.int32, (VP, B), 0)
        anchor = jnp.where(iota_v < V, anchor, NEG_INF)
        colmax = jnp.max(anchor, axis=0, keepdims=True)
        idx_ref[...] = jnp.min(
            jnp.where(anchor == colmax, iota_v, V), axis=0, keepdims=True)

        m0m = m0_ref[...]
        m1m = m1_ref[...]
        mcol = jnp.maximum(jnp.max(m0m, axis=0, keepdims=True),
                           jnp.max(m1m, axis=0, keepdims=True))
        e0 = jnp.exp(m0m - mcol)
        e1 = jnp.exp(m1m - mcol)
        lanes = jax.lax.broadcasted_iota(jnp.int32, (B, B), 1)
        rows = jax.lax.broadcasted_iota(jnp.int32, (B, B), 0)
        diag = lanes == rows
        z = (jnp.sum(e0, axis=0) + jnp.sum(e1, axis=0)
             - jnp.sum(jnp.where(diag, e1, 0.0), axis=0))
        logz = jnp.log(z) + mcol[0]
        diag0 = jnp.sum(jnp.where(diag, m0m, 0.0), axis=0)
        loss_ref[...] = (jnp.sum(logz - diag0) * (1.0 / B)).reshape(1, 1)


def _gather_kernel(idx_ref, vis_ref, emb_ref, sem):
    copies = []
    for b in range(B):
        c = pltpu.make_async_copy(
            vis_ref.at[pl.ds(idx_ref[b], 1), b, :], emb_ref.at[b], sem)
        c.start()
        copies.append(c)
    for c in copies:
        c.wait()


@jax.jit
def kernel(fusion_fs, lan_fs):
    vis_t = jnp.transpose(fusion_fs, (1, 0, 2))
    lmat = lan_fs.reshape(B, D)

    loss2d, idx2d = pl.pallas_call(
        _sim_kernel,
        grid=(NSTEPS,),
        in_specs=[
            pl.BlockSpec((B, D), lambda i: (0, 0)),
            pl.BlockSpec((TV, B, D), lambda i: (i, 0, 0)),
        ],
        out_specs=[
            pl.BlockSpec((1, 1), lambda i: (0, 0)),
            pl.BlockSpec((1, B), lambda i: (0, 0)),
        ],
        out_shape=[
            jax.ShapeDtypeStruct((1, 1), jnp.float32),
            jax.ShapeDtypeStruct((1, B), jnp.int32),
        ],
        scratch_shapes=[
            pltpu.VMEM((B, B), jnp.float32),
            pltpu.VMEM((B, B), jnp.float32),
            pltpu.VMEM((VP, D), jnp.float32),
        ],
    )(lmat, vis_t)

    emb = pl.pallas_call(
        _gather_kernel,
        grid_spec=pltpu.PrefetchScalarGridSpec(
            num_scalar_prefetch=1,
            grid=(1,),
            in_specs=[pl.BlockSpec(memory_space=pl.ANY)],
            out_specs=pl.BlockSpec((B, 1, D), lambda i, idx: (0, 0, 0)),
            scratch_shapes=[pltpu.SemaphoreType.DMA],
        ),
        out_shape=jax.ShapeDtypeStruct((B, 1, D), jnp.float32),
    )(idx2d[0], vis_t)

    return loss2d[0, 0], emb
